# baseline (speedup 1.0000x reference)
import functools

import jax
import jax.numpy as jnp
from jax import lax
from jax.experimental import pallas as pl
from jax.experimental.pallas import tpu as pltpu
from jax.experimental.pallas import tpu_sc as plsc

L = 16


@functools.lru_cache(maxsize=None)
def _build(B, NEG, V, D):
    info = plsc.get_sparse_core_info()
    NC, NS = info.num_cores, info.num_subcores
    NW = NC * NS
    K = 2 + NEG
    CH = 8
    ROWS = CH * K
    D2 = 2 * D
    BW = B // NW
    n_chunks = BW // CH
    NSUP = n_chunks // 3
    NTV = D // L
    TAIL = D - NTV * L
    assert B % NW == 0 and V % 2 == 0 and n_chunks == 3 * NSUP + 1

    mesh = plsc.VectorSubcoreMesh(core_axis_name="c", subcore_axis_name="s")

    @functools.partial(
        pl.kernel,
        mesh=mesh,
        compiler_params=pltpu.CompilerParams(
            use_tc_tiling_on_sc=False, needs_layout_passes=False),
        out_type=jax.ShapeDtypeStruct((B * K, D), jnp.float32),
        scratch_types=[
            pltpu.VMEM((BW * K,), jnp.int32),
            pltpu.VMEM((BW * K,), jnp.int32),
            pltpu.VMEM((BW * 8,), jnp.int32),
            pltpu.VMEM((ROWS, D), jnp.float32),
            pltpu.VMEM((ROWS, D), jnp.float32),
            pltpu.VMEM((ROWS, D2), jnp.float32),
            pltpu.VMEM((ROWS, D), jnp.float32),
            pltpu.SemaphoreType.DMA,
            pltpu.SemaphoreType.DMA,
            pltpu.SemaphoreType.DMA,
            pltpu.SemaphoreType.DMA,
            pltpu.SemaphoreType.DMA,
            pltpu.SemaphoreType.DMA,
        ],
    )
    def skipgram(comb_hbm, gidx_hbm, bc_hbm, wt_hbm, wc_hbm, wt2_hbm, wc2_hbm,
                 out_hbm, comb_w, gidx_w, bc_w, bufA0, bufA1, pairs, outbB,
                 gsemA0, gsemA1, wsemA0, wsemA1, gsemB, wsemB):
        wid = lax.axis_index("s") * NC + lax.axis_index("c")
        wrow0 = wid * (BW * K)
        iota = lax.iota(jnp.int32, L)

        pltpu.sync_copy(comb_hbm.at[pl.ds(wrow0, BW * K)], comb_w)
        pltpu.sync_copy(gidx_hbm.at[pl.ds(wrow0, BW * K)], gidx_w)
        pltpu.sync_copy(bc_hbm.at[pl.ds(wid * (BW * 8), BW * 8)], bc_w)

        def a_chunk(t, c, buf, gsem, wsem):
            row0 = wrow0 + c * ROWS
            lb = pl.multiple_of(c * ROWS, 8)

            @pl.when(t >= 1)
            def _():
                pltpu.make_async_copy(
                    buf, out_hbm.at[pl.ds(row0, ROWS)], wsem).wait()

            vecs = [comb_w[pl.ds(lb, L)], comb_w[pl.ds(lb + L, L)],
                    comb_w[pl.ds(lb + 2 * L, L)],
                    comb_w[pl.ds(lb + ROWS - L, L)]]
            for k in range(ROWS):
                if k < 3 * L:
                    vec, j = vecs[k // L], k % L
                else:
                    vec, j = vecs[3], k - (ROWS - L)
                src = wt_hbm if k % K == 0 else wc_hbm
                pltpu.make_async_copy(
                    src.at[pl.ds(vec[j], 1)], buf.at[pl.ds(k, 1)], gsem).start()

            pltpu.make_async_copy(
                wt_hbm.at[pl.ds(0, ROWS)], buf, gsem).wait()
            pltpu.make_async_copy(
                buf, out_hbm.at[pl.ds(row0, ROWS)], wsem).start()

        def b_issue(c):
            lb = pl.multiple_of(c * ROWS, 8)
            pltpu.make_async_copy(
                wt2_hbm.at[gidx_w.at[pl.ds(lb, CH)]],
                pairs.at[pl.ds(0, CH)], gsemB).start()
            pltpu.make_async_copy(
                wc2_hbm.at[gidx_w.at[pl.ds(lb + CH, ROWS - CH)]],
                pairs.at[pl.ds(CH, ROWS - CH)], gsemB).start()

        def b_chunk(t, c):
            row0 = wrow0 + c * ROWS
            bcb = pl.multiple_of(c * (CH * 8), 8)

            @pl.when(t >= 1)
            def _():
                pltpu.make_async_copy(
                    outbB, out_hbm.at[pl.ds(row0, ROWS)], wsemB).wait()

            pltpu.make_async_copy(
                wt2_hbm.at[pl.ds(0, CH)], pairs.at[pl.ds(0, CH)], gsemB).wait()
            pltpu.make_async_copy(
                wt2_hbm.at[pl.ds(0, ROWS - CH)],
                pairs.at[pl.ds(CH, ROWS - CH)], gsemB).wait()

            exts = [bc_w[pl.ds(bcb + v * L, L)] for v in range(CH * 8 // L)]
            for k in range(ROWS):
                i, j = k // K, k % K
                if j == 0:
                    srcrow = i
                elif j == 1:
                    srcrow = CH + i
                else:
                    srcrow = 2 * CH + i * NEG + (j - 2)
                rowv = jnp.full((L,), srcrow, jnp.int32)
                base = exts[i // 2][(i % 2) * 8 + j]
                for cc in range(NTV):
                    colv = base + cc * L + iota
                    x = plsc.load_gather(pairs, [rowv, colv])
                    outbB[k, pl.ds(cc * L, L)] = x
                colv = jnp.minimum(base + NTV * L + iota, base + D - 1)
                x = plsc.load_gather(pairs, [rowv, colv])
                plsc.store_scatter(
                    outbB, [jnp.full((L,), k, jnp.int32), NTV * L + iota],
                    x, mask=iota < TAIL)

            pltpu.make_async_copy(
                outbB, out_hbm.at[pl.ds(row0, ROWS)], wsemB).start()

            @pl.when(t + 1 < NSUP)
            def _():
                b_issue(3 * (t + 1) + 2)

        b_issue(2)

        def loop_body(t, carry):
            a_chunk(t, 3 * t, bufA0, gsemA0, wsemA0)
            a_chunk(t, 3 * t + 1, bufA1, gsemA1, wsemA1)
            b_chunk(t, 3 * t + 2)
            return carry

        lax.fori_loop(0, NSUP, loop_body, 0)

        a_chunk(NSUP, n_chunks - 1, bufA0, gsemA0, wsemA0)

        pltpu.make_async_copy(
            bufA0, out_hbm.at[pl.ds(wrow0, ROWS)], wsemA0).wait()
        pltpu.make_async_copy(
            bufA1, out_hbm.at[pl.ds(wrow0, ROWS)], wsemA1).wait()
        pltpu.make_async_copy(
            outbB, out_hbm.at[pl.ds(wrow0, ROWS)], wsemB).wait()

    return skipgram


def kernel(target_words, context_words, negative_examples, W_target, W_context):
    B = target_words.shape[0]
    NEG = negative_examples.shape[1]
    V, D = W_target.shape
    K = 2 + NEG
    CH = 8
    tw = target_words.astype(jnp.int32)
    cw = context_words.astype(jnp.int32)
    ne = negative_examples.astype(jnp.int32)

    comb = jnp.concatenate([tw[:, None], cw[:, None], ne], axis=1)
    comb_f = comb.reshape(B * K)
    gidx = jnp.concatenate(
        [(tw >> 1).reshape(B // CH, CH),
         (cw >> 1).reshape(B // CH, CH),
         (ne >> 1).reshape(B // CH, CH * NEG)], axis=1).reshape(B * K)
    bc = (comb & 1) * D
    bc = jnp.concatenate([bc, jnp.zeros((B, 1), jnp.int32)], axis=1)
    bc = bc.reshape(B * 8)

    wt2 = W_target.reshape(V // 2, 2 * D)
    wc2 = W_context.reshape(V // 2, 2 * D)
    fn = _build(B, NEG, V, D)
    out = fn(comb_f, gidx, bc, W_target, W_context, wt2, wc2)
    return out.reshape(B, K, D)
